# Initial kernel scaffold; baseline (speedup 1.0000x reference)
import jax
import jax.numpy as jnp
from jax.experimental import pallas as pl


def kernel(x, edge_index, gin_eps, W1, b1, gamma1, beta1, W2, b2, gamma4, beta4, Wl1, bl1, Wl3, bl3):
    raise NotImplementedError("write your pallas kernel here")


# speedup vs baseline: 4.8856x; 4.8856x over previous
import functools

import jax
import jax.numpy as jnp
from jax import lax
from jax.experimental import pallas as pl
from jax.experimental.pallas import tpu as pltpu
from jax.experimental.pallas import tpu_sc as plsc

N = 10000
H = 128
E = 320000
NC = 2
NS = 16
NW = NC * NS
EPW = E // NW
CHUNK = 80
NCHUNK = EPW // CHUNK
RPT = (N // NS) // 8 * 8
RTAIL = N - RPT * NS

assert EPW * NW == E and NCHUNK * CHUNK == EPW and RTAIL % 8 == 0


def _make_sc_agg():
    mesh = plsc.VectorSubcoreMesh(core_axis_name="c", subcore_axis_name="s",
                                  num_cores=NC, num_subcores=NS)

    @functools.partial(
        pl.kernel,
        out_type=jax.ShapeDtypeStruct((NC, N, H), jnp.float32),
        mesh=mesh,
        scratch_types=[
            pltpu.VMEM_SHARED((N, H), jnp.float32),
            pltpu.VMEM((CHUNK,), jnp.int32),
            pltpu.VMEM((CHUNK,), jnp.int32),
            pltpu.VMEM((CHUNK, H), jnp.float32),
            pltpu.SemaphoreType.DMA,
        ],
    )
    def sc_agg(x_hbm, src_hbm, dst_hbm, zeros_hbm, out_hbm,
               acc_sh, src_v, dst_v, rows_v, sem):
        c = lax.axis_index("c")
        s = lax.axis_index("s")
        wid = s * NC + c

        pltpu.sync_copy(zeros_hbm.at[pl.ds(0, RPT)], acc_sh.at[pl.ds(s * RPT, RPT)])

        @pl.when(s == NS - 1)
        def _():
            pltpu.sync_copy(zeros_hbm.at[pl.ds(0, RTAIL)],
                            acc_sh.at[pl.ds(NS * RPT, RTAIL)])

        plsc.subcore_barrier()

        base = wid * EPW

        def chunk_body(j, carry):
            off = base + j * CHUNK
            pltpu.sync_copy(src_hbm.at[pl.ds(off, CHUNK)], src_v)
            pltpu.sync_copy(dst_hbm.at[pl.ds(off, CHUNK)], dst_v)
            pltpu.async_copy(x_hbm.at[src_v], rows_v, sem).wait()
            pltpu.sync_copy(rows_v, acc_sh.at[dst_v], add=True)
            return carry

        lax.fori_loop(0, NCHUNK, chunk_body, 0, unroll=False)

        plsc.subcore_barrier()
        pltpu.sync_copy(acc_sh.at[pl.ds(s * RPT, RPT)],
                        out_hbm.at[c, pl.ds(s * RPT, RPT)])

        @pl.when(s == NS - 1)
        def _():
            pltpu.sync_copy(acc_sh.at[pl.ds(NS * RPT, RTAIL)],
                            out_hbm.at[c, pl.ds(NS * RPT, RTAIL)])

    return sc_agg


_SC_AGG_CACHE = []


def _sc_agg(*args):
    if not _SC_AGG_CACHE:
        _SC_AGG_CACHE.append(_make_sc_agg())
    return _SC_AGG_CACHE[0](*args)


def _tc_head_body(eps_ref, x_ref, agg_ref, w1_ref, b1_ref, g1_ref, be1_ref,
                  w2_ref, b2_ref, g4_ref, be4_ref, wl1_ref, bl1_ref,
                  wl3_ref, bl3_ref, out_ref):
    eps = eps_ref[0, 0]
    x = x_ref[...]
    agg = agg_ref[0] + agg_ref[1]

    h = (1.0 + eps) * x + agg
    h = jnp.dot(h, w1_ref[...], preferred_element_type=jnp.float32) + b1_ref[...]
    m = jnp.mean(h, axis=0, keepdims=True)
    v = jnp.mean((h - m) * (h - m), axis=0, keepdims=True)
    h = g1_ref[...] * (h - m) * lax.rsqrt(v + 1e-5) + be1_ref[...]
    h = jnp.maximum(h, 0.0)
    h = jnp.dot(h, w2_ref[...], preferred_element_type=jnp.float32) + b2_ref[...]
    h = jnp.where(h > 0, h, 0.01 * h)
    m4 = jnp.mean(h, axis=0, keepdims=True)
    v4 = jnp.mean((h - m4) * (h - m4), axis=0, keepdims=True)
    h = g4_ref[...] * (h - m4) * lax.rsqrt(v4 + 1e-5) + be4_ref[...]
    h = jnp.where(h > 0, h, 0.1 * h)
    h = x + 0.01 * h
    h = jnp.dot(h, wl1_ref[...], preferred_element_type=jnp.float32) + bl1_ref[...]
    h = jnp.where(h > 0, h, 0.1 * h)
    out_ref[...] = (jnp.dot(h, wl3_ref[...], preferred_element_type=jnp.float32)
                    + bl3_ref[...])


def _tc_head(gin_eps, x, agg2, W1, b1, gamma1, beta1, W2, b2, gamma4, beta4,
             Wl1, bl1, Wl3, bl3):
    C = Wl3.shape[1]
    eps_arr = jnp.reshape(gin_eps, (1, 1))
    smem_spec = pl.BlockSpec(memory_space=pltpu.SMEM)
    return pl.pallas_call(
        _tc_head_body,
        out_shape=jax.ShapeDtypeStruct((N, C), jnp.float32),
        in_specs=[smem_spec] + [pl.BlockSpec(memory_space=pltpu.VMEM)] * 14,
        out_specs=pl.BlockSpec(memory_space=pltpu.VMEM),
    )(eps_arr, x, agg2,
      W1, jnp.reshape(b1, (1, H)), jnp.reshape(gamma1, (1, H)),
      jnp.reshape(beta1, (1, H)),
      W2, jnp.reshape(b2, (1, H)), jnp.reshape(gamma4, (1, H)),
      jnp.reshape(beta4, (1, H)),
      Wl1, jnp.reshape(bl1, (1, Wl1.shape[1])),
      Wl3, jnp.reshape(bl3, (1, C)))


def kernel(x, edge_index, gin_eps, W1, b1, gamma1, beta1, W2, b2,
           gamma4, beta4, Wl1, bl1, Wl3, bl3):
    src = edge_index[0].astype(jnp.int32)
    dst = edge_index[1].astype(jnp.int32)
    zeros = jnp.zeros((RPT, H), dtype=jnp.float32)
    agg2 = _sc_agg(x, src, dst, zeros)
    return _tc_head(gin_eps, x, agg2, W1, b1, gamma1, beta1, W2, b2,
                    gamma4, beta4, Wl1, bl1, Wl3, bl3)
